# baseline (speedup 1.0000x reference)
import functools

import jax
import jax.numpy as jnp
from jax import lax
from jax.experimental import pallas as pl
from jax.experimental.pallas import tpu as pltpu
from jax.experimental.pallas import tpu_sc as plsc

N = 10000
E = 320000
D = 128
DH = D // 2
EPSILON = 1e-5

EPAD = 327680
CHUNK = 128
NCHUNK = EPAD // 16 // CHUNK
NB = 4
NI = 8
GA = 2
NROWS = 10240
RPS = NROWS // 16


def _sc_body(feat_hbm, ei_hbm, out_hbm, ibuf, rows, agg_s, isem, gsem, ssem):
    c = lax.axis_index("c")
    s = lax.axis_index("s")

    def start_idx(g, ib):
        pltpu.async_copy(ei_hbm.at[c, s, g], ibuf.at[ib], isem)

    def wait_idx(g, ib):
        pltpu.make_async_copy(ei_hbm.at[c, s, g], ibuf.at[ib], isem).wait()

    def start_gather(ib, b):
        pltpu.async_copy(feat_hbm.at[ibuf.at[ib, 0]], rows.at[b], gsem)

    def wait_gather(ib, b):
        pltpu.make_async_copy(feat_hbm.at[ibuf.at[ib, 0]], rows.at[b], gsem).wait()

    def start_scatter(ib, b):
        pltpu.async_copy(rows.at[b], agg_s.at[ibuf.at[ib, 1]], ssem, add=True)

    def wait_scatter(ib, b):
        pltpu.make_async_copy(rows.at[b], agg_s.at[ibuf.at[ib, 1]], ssem).wait()

    for g in range(NI - GA):
        start_idx(g, g)

    zero16 = jnp.zeros((16,), jnp.float32)

    def _zrow(i, carry):
        for j in range(DH // 16):
            rows[NB - 1, i, pl.ds(j * 16, 16)] = zero16
        return carry

    lax.fori_loop(0, CHUNK, _zrow, 0)
    for k in range(RPS // CHUNK):
        pltpu.sync_copy(rows.at[NB - 1],
                        agg_s.at[pl.ds(s * RPS + k * CHUNK, CHUNK)])
    plsc.subcore_barrier()

    def step(g, slot, first=False, do_idx=True, do_gather=True):
        if not first:
            wait_scatter((slot - GA) % NI, (slot - GA) % NB)
        if do_idx:
            start_idx(g + NI - GA, (slot - GA) % NI)
        if do_gather:
            wait_idx(g + GA, (slot + GA) % NI)
            start_gather((slot + GA) % NI, (slot + GA) % NB)
        wait_gather(slot % NI, slot % NB)
        start_scatter(slot % NI, slot % NB)

    for g in range(GA):
        wait_idx(g, g)
        start_gather(g, g)

    for g in range(NI):
        step(g, g, first=(g < GA))

    def _main(t, carry):
        for b in range(NI):
            step(NI * t + b, b)
        return carry

    lax.fori_loop(1, NCHUNK // NI - 1, _main, 0)

    for g in range(NCHUNK - NI, NCHUNK):
        step(g, g % NI,
             do_idx=(g + NI - GA < NCHUNK), do_gather=(g + GA < NCHUNK))
    for g in range(NCHUNK - GA, NCHUNK):
        wait_scatter(g % NI, g % NB)

    plsc.subcore_barrier()

    for k in range(RPS // CHUNK):
        r0 = s * RPS + k * CHUNK
        pltpu.sync_copy(agg_s.at[pl.ds(r0, CHUNK)], out_hbm.at[c, pl.ds(r0, CHUNK)])


_sc_aggregate = functools.partial(
    pl.kernel,
    mesh=plsc.VectorSubcoreMesh(core_axis_name="c", subcore_axis_name="s"),
    compiler_params=pltpu.CompilerParams(use_tc_tiling_on_sc=False),
    out_type=jax.ShapeDtypeStruct((2, NROWS, DH), jnp.float32),
    scratch_types=[
        pltpu.VMEM((NI, 2, CHUNK), jnp.int32),
        pltpu.VMEM((NB, CHUNK, DH), jnp.float32),
        pltpu.VMEM_SHARED((NROWS, DH), jnp.float32),
        pltpu.SemaphoreType.DMA,
        pltpu.SemaphoreType.DMA,
        pltpu.SemaphoreType.DMA,
    ],
)(_sc_body)


def _tc_body(p_ref, w_ref, g_ref, b_ref, o_ref):
    a = jnp.concatenate(
        [p_ref[0, pl.ds(0, N), :], p_ref[1, pl.ds(0, N), :]], axis=1)
    agg = lax.dot_general(
        a, w_ref[...], (((1,), (1,)), ((), ())),
        preferred_element_type=jnp.float32,
        precision=lax.Precision.HIGHEST,
    )
    mean = jnp.mean(agg, axis=0, keepdims=True)
    cent = agg - mean
    var = jnp.mean(cent * cent, axis=0, keepdims=True)
    inv = lax.rsqrt(var + EPSILON)
    o_ref[...] = jnp.maximum(cent * inv * g_ref[...] + b_ref[...], 0.0)


def kernel(feature, edge_index, W, gamma, beta):
    src = edge_index[0]
    dst = edge_index[1]
    npad = EPAD - E
    src_p = jnp.concatenate([src, jnp.zeros((npad,), jnp.int32)])
    dst_p = jnp.concatenate([dst, jnp.full((npad,), NROWS - 1, jnp.int32)])
    feat_t = feature.reshape(2 * N, DH)
    src_r = src_p.reshape(16, NCHUNK, CHUNK)
    dst_r = dst_p.reshape(16, NCHUNK, CHUNK)
    ei = jnp.stack([jnp.stack([2 * src_r + cc, dst_r], axis=2)
                    for cc in range(2)], axis=0)

    partial = _sc_aggregate(feat_t, ei)

    out = pl.pallas_call(
        _tc_body,
        out_shape=jax.ShapeDtypeStruct((N, D), jnp.float32),
    )(partial, W, gamma.reshape(1, D), beta.reshape(1, D))
    return out


# speedup vs baseline: 1.1523x; 1.1523x over previous
import functools

import jax
import jax.numpy as jnp
from jax import lax
from jax.experimental import pallas as pl
from jax.experimental.pallas import tpu as pltpu
from jax.experimental.pallas import tpu_sc as plsc

N = 10000
E = 320000
D = 128
DH = D // 2
EPSILON = 1e-5

EPAD = 327680
CHUNK = 128
NCHUNK = EPAD // 16 // CHUNK
NB = 4
NI = 8
GA = 2
NROWS = 10240
RPS = NROWS // 16


def _sc_body(feat_hbm, ei_hbm, out_hbm, ibuf, rows, agg_s, isem, gsem, ssem):
    c = lax.axis_index("c")
    s = lax.axis_index("s")

    def start_idx(g, ib):
        pltpu.async_copy(ei_hbm.at[c, s, g], ibuf.at[ib], isem)

    def wait_idx(g, ib):
        pltpu.make_async_copy(ei_hbm.at[c, s, g], ibuf.at[ib], isem).wait()

    def start_gather(ib, b):
        pltpu.async_copy(feat_hbm.at[ibuf.at[ib, 0]], rows.at[b], gsem)

    def wait_gather(ib, b):
        pltpu.make_async_copy(feat_hbm.at[ibuf.at[ib, 0]], rows.at[b], gsem).wait()

    def start_scatter(ib, b):
        pltpu.async_copy(rows.at[b], agg_s.at[ibuf.at[ib, 1]], ssem, add=True)

    def wait_scatter(ib, b):
        pltpu.make_async_copy(rows.at[b], agg_s.at[ibuf.at[ib, 1]], ssem).wait()

    for g in range(NI - GA):
        start_idx(g, g)

    zero16 = jnp.zeros((16,), jnp.float32)

    def _zrow(i, carry):
        for j in range(DH // 16):
            rows[NB - 1, i, pl.ds(j * 16, 16)] = zero16
        return carry

    lax.fori_loop(0, CHUNK, _zrow, 0)
    for k in range(RPS // CHUNK):
        pltpu.sync_copy(rows.at[NB - 1],
                        agg_s.at[pl.ds(s * RPS + k * CHUNK, CHUNK)])
    plsc.subcore_barrier()

    def step(g, slot, first=False, do_idx=True, do_gather=True):
        if not first:
            wait_scatter((slot - GA) % NI, (slot - GA) % NB)
        if do_idx:
            start_idx(g + NI - GA, (slot - GA) % NI)
        if do_gather:
            wait_idx(g + GA, (slot + GA) % NI)
            start_gather((slot + GA) % NI, (slot + GA) % NB)
        wait_gather(slot % NI, slot % NB)
        start_scatter(slot % NI, slot % NB)

    for g in range(GA):
        wait_idx(g, g)
        start_gather(g, g)

    for g in range(NI):
        step(g, g, first=(g < GA))

    def _main(t, carry):
        for b in range(NI):
            step(NI * t + b, b)
        return carry

    lax.fori_loop(1, NCHUNK // NI - 1, _main, 0)

    for g in range(NCHUNK - NI, NCHUNK):
        step(g, g % NI,
             do_idx=(g + NI - GA < NCHUNK), do_gather=(g + GA < NCHUNK))
    for g in range(NCHUNK - GA, NCHUNK):
        wait_scatter(g % NI, g % NB)

    plsc.subcore_barrier()

    for k in range(RPS // CHUNK):
        r0 = s * RPS + k * CHUNK
        pltpu.sync_copy(agg_s.at[pl.ds(r0, CHUNK)], out_hbm.at[c, pl.ds(r0, CHUNK)])


_sc_aggregate = functools.partial(
    pl.kernel,
    mesh=plsc.VectorSubcoreMesh(core_axis_name="c", subcore_axis_name="s"),
    compiler_params=pltpu.CompilerParams(use_tc_tiling_on_sc=False),
    out_type=jax.ShapeDtypeStruct((2, NROWS, DH), jnp.float32),
    scratch_types=[
        pltpu.VMEM((NI, 2, CHUNK), jnp.int32),
        pltpu.VMEM((NB, CHUNK, DH), jnp.float32),
        pltpu.VMEM_SHARED((NROWS, DH), jnp.float32),
        pltpu.SemaphoreType.DMA,
        pltpu.SemaphoreType.DMA,
        pltpu.SemaphoreType.DMA,
    ],
)(_sc_body)


def _tc_body(p_ref, w_ref, g_ref, b_ref, o_ref):
    a = jnp.concatenate(
        [p_ref[0, pl.ds(0, N), :], p_ref[1, pl.ds(0, N), :]], axis=1)
    agg = lax.dot_general(
        a, w_ref[...], (((1,), (1,)), ((), ())),
        preferred_element_type=jnp.float32,
        precision=lax.Precision.HIGHEST,
    )
    mean = jnp.mean(agg, axis=0, keepdims=True)
    cent = agg - mean
    var = jnp.mean(cent * cent, axis=0, keepdims=True)
    inv = lax.rsqrt(var + EPSILON)
    o_ref[...] = jnp.maximum(cent * inv * g_ref[...] + b_ref[...], 0.0)


def kernel(feature, edge_index, W, gamma, beta):
    src = edge_index[0]
    dst = edge_index[1]
    npad = EPAD - E
    src_p = jnp.concatenate([src, jnp.zeros((npad,), jnp.int32)])
    dst_p = jnp.concatenate([dst, jnp.full((npad,), NROWS - 1, jnp.int32)])
    feat_t = feature.reshape(N, 2, DH).transpose(1, 0, 2).reshape(2 * N, DH)
    src_r = src_p.reshape(16, NCHUNK, CHUNK)
    dst_r = dst_p.reshape(16, NCHUNK, CHUNK)
    ei = jnp.stack([jnp.stack([src_r + cc * N, dst_r], axis=2)
                    for cc in range(2)], axis=0)

    partial = _sc_aggregate(feat_t, ei)

    out = pl.pallas_call(
        _tc_body,
        out_shape=jax.ShapeDtypeStruct((N, D), jnp.float32),
    )(partial, W, gamma.reshape(1, D), beta.reshape(1, D))
    return out


# speedup vs baseline: 1.2571x; 1.0910x over previous
import functools

import jax
import jax.numpy as jnp
from jax import lax
from jax.experimental import pallas as pl
from jax.experimental.pallas import tpu as pltpu
from jax.experimental.pallas import tpu_sc as plsc

N = 10000
E = 320000
D = 128
DH = D // 2
EPSILON = 1e-5

EPAD = 327680
CHUNK = 128
NCHUNK = EPAD // 16 // CHUNK
NB = 5
NI = 10
GA = 2
SD = NB - GA
NROWS = 10240
RPS = NROWS // 16


def _sc_body(feat_hbm, ei_hbm, out_hbm, ibuf, rows, agg_s, isem, gsem, ssem):
    c = lax.axis_index("c")
    s = lax.axis_index("s")

    def start_idx(g, ib):
        pltpu.async_copy(ei_hbm.at[c, s, g], ibuf.at[ib], isem)

    def wait_idx(g, ib):
        pltpu.make_async_copy(ei_hbm.at[c, s, g], ibuf.at[ib], isem).wait()

    def start_gather(ib, b):
        pltpu.async_copy(feat_hbm.at[ibuf.at[ib, 0]], rows.at[b], gsem)

    def wait_gather(ib, b):
        pltpu.make_async_copy(feat_hbm.at[ibuf.at[ib, 0]], rows.at[b], gsem).wait()

    def start_scatter(ib, b):
        pltpu.async_copy(rows.at[b], agg_s.at[ibuf.at[ib, 1]], ssem, add=True)

    def wait_scatter(ib, b):
        pltpu.make_async_copy(rows.at[b], agg_s.at[ibuf.at[ib, 1]], ssem).wait()

    for g in range(NI - SD):
        start_idx(g, g)

    zero16 = jnp.zeros((16,), jnp.float32)

    def _zrow(i, carry):
        for j in range(DH // 16):
            rows[NB - 1, i, pl.ds(j * 16, 16)] = zero16
        return carry

    lax.fori_loop(0, CHUNK, _zrow, 0)
    for k in range(RPS // CHUNK):
        pltpu.sync_copy(rows.at[NB - 1],
                        agg_s.at[pl.ds(s * RPS + k * CHUNK, CHUNK)])
    plsc.subcore_barrier()

    def step(g, slot, first=False, do_idx=True, do_gather=True):
        if not first:
            wait_scatter((slot - SD) % NI, (slot - SD) % NB)
        if do_idx:
            start_idx(g + NI - SD, (slot - SD) % NI)
        if do_gather:
            wait_idx(g + GA, (slot + GA) % NI)
            start_gather((slot + GA) % NI, (slot + GA) % NB)
        wait_gather(slot % NI, slot % NB)
        start_scatter(slot % NI, slot % NB)

    for g in range(GA):
        wait_idx(g, g)
        start_gather(g, g)

    for g in range(NI):
        step(g, g, first=(g < SD))

    def _main(t, carry):
        for b in range(NI):
            step(NI * t + b, b)
        return carry

    lax.fori_loop(1, NCHUNK // NI - 1, _main, 0)

    for g in range(NCHUNK - NI, NCHUNK):
        step(g, g % NI,
             do_idx=(g + NI - SD < NCHUNK), do_gather=(g + GA < NCHUNK))
    for g in range(NCHUNK - SD, NCHUNK):
        wait_scatter(g % NI, g % NB)

    plsc.subcore_barrier()

    for k in range(RPS // CHUNK):
        r0 = s * RPS + k * CHUNK
        pltpu.async_copy(agg_s.at[pl.ds(r0, CHUNK)], out_hbm.at[c, pl.ds(r0, CHUNK)], gsem)
    for k in range(RPS // CHUNK):
        r0 = s * RPS + k * CHUNK
        pltpu.make_async_copy(agg_s.at[pl.ds(r0, CHUNK)],
                              out_hbm.at[c, pl.ds(r0, CHUNK)], gsem).wait()


_sc_aggregate = functools.partial(
    pl.kernel,
    mesh=plsc.VectorSubcoreMesh(core_axis_name="c", subcore_axis_name="s"),
    compiler_params=pltpu.CompilerParams(use_tc_tiling_on_sc=False),
    out_type=jax.ShapeDtypeStruct((2, NROWS, DH), jnp.float32),
    scratch_types=[
        pltpu.VMEM((NI, 2, CHUNK), jnp.int32),
        pltpu.VMEM((NB, CHUNK, DH), jnp.float32),
        pltpu.VMEM_SHARED((NROWS, DH), jnp.float32),
        pltpu.SemaphoreType.DMA,
        pltpu.SemaphoreType.DMA,
        pltpu.SemaphoreType.DMA,
    ],
)(_sc_body)


def _tc_body(p_ref, w_ref, g_ref, b_ref, o_ref):
    a = jnp.concatenate(
        [p_ref[0, pl.ds(0, N), :], p_ref[1, pl.ds(0, N), :]], axis=1)
    agg = lax.dot_general(
        a, w_ref[...], (((1,), (1,)), ((), ())),
        preferred_element_type=jnp.float32,
        precision=lax.Precision.HIGHEST,
    )
    mean = jnp.mean(agg, axis=0, keepdims=True)
    cent = agg - mean
    var = jnp.mean(cent * cent, axis=0, keepdims=True)
    inv = lax.rsqrt(var + EPSILON)
    o_ref[...] = jnp.maximum(cent * inv * g_ref[...] + b_ref[...], 0.0)


def kernel(feature, edge_index, W, gamma, beta):
    src = edge_index[0]
    dst = edge_index[1]
    npad = EPAD - E
    src_p = jnp.concatenate([src, jnp.zeros((npad,), jnp.int32)])
    dst_p = jnp.concatenate([dst, jnp.full((npad,), NROWS - 1, jnp.int32)])
    feat_t = jnp.zeros((2, NROWS, DH), jnp.float32)
    feat_t = feat_t.at[:, :N, :].set(
        feature.reshape(N, 2, DH).transpose(1, 0, 2))
    feat_t = feat_t.reshape(2 * NROWS, DH)
    src_r = src_p.reshape(16, NCHUNK, CHUNK)
    dst_r = dst_p.reshape(16, NCHUNK, CHUNK)
    ei = jnp.stack([jnp.stack([src_r + cc * NROWS, dst_r], axis=2)
                    for cc in range(2)], axis=0)

    partial = _sc_aggregate(feat_t, ei)

    out = pl.pallas_call(
        _tc_body,
        out_shape=jax.ShapeDtypeStruct((N, D), jnp.float32),
    )(partial, W, gamma.reshape(1, D), beta.reshape(1, D))
    return out


# speedup vs baseline: 1.2674x; 1.0082x over previous
import functools

import jax
import jax.numpy as jnp
from jax import lax
from jax.experimental import pallas as pl
from jax.experimental.pallas import tpu as pltpu
from jax.experimental.pallas import tpu_sc as plsc

N = 10000
E = 320000
D = 128
DH = D // 2
EPSILON = 1e-5

EPAD = 327680
CHUNK = 128
NCHUNK = EPAD // 16 // CHUNK
NB = 5
NI = 10
GA = 2
SD = NB - GA
NROWS = 10240
RPS = NROWS // 16


def _sc_body(feat_hbm, ei_hbm, out_hbm, ibuf, rows, agg_s, isem, gsem, ssem):
    c = lax.axis_index("c")
    s = lax.axis_index("s")

    cvec = jnp.zeros((16,), jnp.int32) + c * NROWS

    def start_idx(g, ib):
        pltpu.async_copy(ei_hbm.at[s, g], ibuf.at[ib], isem)

    def wait_idx(g, ib):
        pltpu.make_async_copy(ei_hbm.at[s, g], ibuf.at[ib], isem).wait()
        for j in range(CHUNK // 16):
            v = ibuf[ib, 0, pl.ds(j * 16, 16)]
            ibuf[ib, 0, pl.ds(j * 16, 16)] = v + cvec

    def start_gather(ib, b):
        pltpu.async_copy(feat_hbm.at[ibuf.at[ib, 0]], rows.at[b], gsem)

    def wait_gather(ib, b):
        pltpu.make_async_copy(feat_hbm.at[ibuf.at[ib, 0]], rows.at[b], gsem).wait()

    def start_scatter(ib, b):
        pltpu.async_copy(rows.at[b], agg_s.at[ibuf.at[ib, 1]], ssem, add=True)

    def wait_scatter(ib, b):
        pltpu.make_async_copy(rows.at[b], agg_s.at[ibuf.at[ib, 1]], ssem).wait()

    for g in range(NI - SD):
        start_idx(g, g)

    zero16 = jnp.zeros((16,), jnp.float32)

    def _zrow(i, carry):
        for j in range(DH // 16):
            rows[NB - 1, i, pl.ds(j * 16, 16)] = zero16
        return carry

    lax.fori_loop(0, CHUNK, _zrow, 0)
    for k in range(RPS // CHUNK):
        pltpu.sync_copy(rows.at[NB - 1],
                        agg_s.at[pl.ds(s * RPS + k * CHUNK, CHUNK)])
    plsc.subcore_barrier()

    def step(g, slot, first=False, do_idx=True, do_gather=True):
        if not first:
            wait_scatter((slot - SD) % NI, (slot - SD) % NB)
        if do_idx:
            start_idx(g + NI - SD, (slot - SD) % NI)
        if do_gather:
            wait_idx(g + GA, (slot + GA) % NI)
            start_gather((slot + GA) % NI, (slot + GA) % NB)
        wait_gather(slot % NI, slot % NB)
        start_scatter(slot % NI, slot % NB)

    for g in range(GA):
        wait_idx(g, g)
        start_gather(g, g)

    for g in range(NI):
        step(g, g, first=(g < SD))

    def _main(t, carry):
        for b in range(NI):
            step(NI * t + b, b)
        return carry

    lax.fori_loop(1, NCHUNK // NI - 1, _main, 0)

    for g in range(NCHUNK - NI, NCHUNK):
        step(g, g % NI,
             do_idx=(g + NI - SD < NCHUNK), do_gather=(g + GA < NCHUNK))
    for g in range(NCHUNK - SD, NCHUNK):
        wait_scatter(g % NI, g % NB)

    plsc.subcore_barrier()

    for k in range(RPS // CHUNK):
        r0 = s * RPS + k * CHUNK
        pltpu.async_copy(agg_s.at[pl.ds(r0, CHUNK)], out_hbm.at[c, pl.ds(r0, CHUNK)], gsem)
    for k in range(RPS // CHUNK):
        r0 = s * RPS + k * CHUNK
        pltpu.make_async_copy(agg_s.at[pl.ds(r0, CHUNK)],
                              out_hbm.at[c, pl.ds(r0, CHUNK)], gsem).wait()


_sc_aggregate = functools.partial(
    pl.kernel,
    mesh=plsc.VectorSubcoreMesh(core_axis_name="c", subcore_axis_name="s"),
    compiler_params=pltpu.CompilerParams(use_tc_tiling_on_sc=False),
    out_type=jax.ShapeDtypeStruct((2, NROWS, DH), jnp.float32),
    scratch_types=[
        pltpu.VMEM((NI, 2, CHUNK), jnp.int32),
        pltpu.VMEM((NB, CHUNK, DH), jnp.float32),
        pltpu.VMEM_SHARED((NROWS, DH), jnp.float32),
        pltpu.SemaphoreType.DMA,
        pltpu.SemaphoreType.DMA,
        pltpu.SemaphoreType.DMA,
    ],
)(_sc_body)


def _tc_body(p_ref, w_ref, g_ref, b_ref, o_ref):
    a = jnp.concatenate(
        [p_ref[0, pl.ds(0, N), :], p_ref[1, pl.ds(0, N), :]], axis=1)
    agg = lax.dot_general(
        a, w_ref[...], (((1,), (1,)), ((), ())),
        preferred_element_type=jnp.float32,
        precision=lax.Precision.HIGHEST,
    )
    mean = jnp.mean(agg, axis=0, keepdims=True)
    cent = agg - mean
    var = jnp.mean(cent * cent, axis=0, keepdims=True)
    inv = lax.rsqrt(var + EPSILON)
    o_ref[...] = jnp.maximum(cent * inv * g_ref[...] + b_ref[...], 0.0)


def kernel(feature, edge_index, W, gamma, beta):
    src = edge_index[0]
    dst = edge_index[1]
    npad = EPAD - E
    src_p = jnp.concatenate([src, jnp.zeros((npad,), jnp.int32)])
    dst_p = jnp.concatenate(
        [dst, N + jnp.arange(npad, dtype=jnp.int32) % (NROWS - N)])
    feat_t = jnp.zeros((2, NROWS, DH), jnp.float32)
    feat_t = feat_t.at[:, :N, :].set(
        feature.reshape(N, 2, DH).transpose(1, 0, 2))
    feat_t = feat_t.reshape(2 * NROWS, DH)
    ei = jnp.stack([src_p.reshape(16, NCHUNK, CHUNK),
                    dst_p.reshape(16, NCHUNK, CHUNK)], axis=2)

    partial = _sc_aggregate(feat_t, ei)

    out = pl.pallas_call(
        _tc_body,
        out_shape=jax.ShapeDtypeStruct((N, D), jnp.float32),
    )(partial, W, gamma.reshape(1, D), beta.reshape(1, D))
    return out


# speedup vs baseline: 1.7176x; 1.3552x over previous
import functools

import jax
import jax.numpy as jnp
from jax import lax
from jax.experimental import pallas as pl
from jax.experimental.pallas import tpu as pltpu
from jax.experimental.pallas import tpu_sc as plsc

N = 10000
E = 320000
D = 128
DH = D // 2
EPSILON = 1e-5

EPAD = 327680
CHUNK = 128
NCHUNK = EPAD // 16 // CHUNK
NB = 5
NI = 10
GA = 2
SD = NB - GA
NROWS = 10240
RPS = NROWS // 16


def _sc_body(feat_hbm, ei_hbm, out_hbm, ibuf, rows, agg_a, agg_b, isem, gsem, ssem):
    c = lax.axis_index("c")
    s = lax.axis_index("s")

    cvec = jnp.zeros((16,), jnp.int32) + c * NROWS

    def start_idx(g, ib):
        pltpu.async_copy(ei_hbm.at[s, g], ibuf.at[ib], isem)

    def wait_idx(g, ib):
        pltpu.make_async_copy(ei_hbm.at[s, g], ibuf.at[ib], isem).wait()
        for j in range(CHUNK // 16):
            v = ibuf[ib, 0, pl.ds(j * 16, 16)]
            ibuf[ib, 0, pl.ds(j * 16, 16)] = v + cvec

    def start_gather(ib, b):
        pltpu.async_copy(feat_hbm.at[ibuf.at[ib, 0]], rows.at[b], gsem)

    def wait_gather(ib, b):
        pltpu.make_async_copy(feat_hbm.at[ibuf.at[ib, 0]], rows.at[b], gsem).wait()

    def _acc(slot):
        return agg_a if slot % 2 == 0 else agg_b

    def start_scatter(ib, b):
        pltpu.async_copy(rows.at[b], _acc(ib).at[ibuf.at[ib, 1]], ssem, add=True)

    def wait_scatter(ib, b):
        pltpu.make_async_copy(rows.at[b], _acc(ib).at[ibuf.at[ib, 1]], ssem).wait()

    for g in range(NI - SD):
        start_idx(g, g)

    zero32 = jnp.zeros((32,), jnp.bfloat16)

    def _zrow(i, carry):
        for j in range(DH // 32):
            rows[NB - 1, i, pl.ds(j * 32, 32)] = zero32
        return carry

    lax.fori_loop(0, CHUNK, _zrow, 0)
    for acc in (agg_a, agg_b):
        for k in range(RPS // CHUNK):
            pltpu.sync_copy(rows.at[NB - 1],
                            acc.at[pl.ds(s * RPS + k * CHUNK, CHUNK)])
    plsc.subcore_barrier()

    def step(g, slot, first=False, do_idx=True, do_gather=True):
        if not first:
            wait_scatter((slot - SD) % NI, (slot - SD) % NB)
        if do_idx:
            start_idx(g + NI - SD, (slot - SD) % NI)
        if do_gather:
            wait_idx(g + GA, (slot + GA) % NI)
            start_gather((slot + GA) % NI, (slot + GA) % NB)
        wait_gather(slot % NI, slot % NB)
        start_scatter(slot % NI, slot % NB)

    for g in range(GA):
        wait_idx(g, g)
        start_gather(g, g)

    for g in range(NI):
        step(g, g, first=(g < SD))

    def _main(t, carry):
        for b in range(NI):
            step(NI * t + b, b)
        return carry

    lax.fori_loop(1, NCHUNK // NI - 1, _main, 0)

    for g in range(NCHUNK - NI, NCHUNK):
        step(g, g % NI,
             do_idx=(g + NI - SD < NCHUNK), do_gather=(g + GA < NCHUNK))
    for g in range(NCHUNK - SD, NCHUNK):
        wait_scatter(g % NI, g % NB)

    plsc.subcore_barrier()

    for a, acc in enumerate((agg_a, agg_b)):
        for k in range(RPS // CHUNK):
            r0 = s * RPS + k * CHUNK
            pltpu.async_copy(acc.at[pl.ds(r0, CHUNK)],
                             out_hbm.at[c, a, pl.ds(r0, CHUNK)], gsem)
    for a, acc in enumerate((agg_a, agg_b)):
        for k in range(RPS // CHUNK):
            r0 = s * RPS + k * CHUNK
            pltpu.make_async_copy(acc.at[pl.ds(r0, CHUNK)],
                                  out_hbm.at[c, a, pl.ds(r0, CHUNK)], gsem).wait()


_sc_aggregate = functools.partial(
    pl.kernel,
    mesh=plsc.VectorSubcoreMesh(core_axis_name="c", subcore_axis_name="s"),
    compiler_params=pltpu.CompilerParams(use_tc_tiling_on_sc=False),
    out_type=jax.ShapeDtypeStruct((2, 2, NROWS, DH), jnp.bfloat16),
    scratch_types=[
        pltpu.VMEM((NI, 2, CHUNK), jnp.int32),
        pltpu.VMEM((NB, CHUNK, DH), jnp.bfloat16),
        pltpu.VMEM_SHARED((NROWS, DH), jnp.bfloat16),
        pltpu.VMEM_SHARED((NROWS, DH), jnp.bfloat16),
        pltpu.SemaphoreType.DMA,
        pltpu.SemaphoreType.DMA,
        pltpu.SemaphoreType.DMA,
    ],
)(_sc_body)


def _tc_body(p_ref, w_ref, g_ref, b_ref, o_ref):
    left = (p_ref[0, 0, pl.ds(0, N), :].astype(jnp.float32)
            + p_ref[0, 1, pl.ds(0, N), :].astype(jnp.float32))
    right = (p_ref[1, 0, pl.ds(0, N), :].astype(jnp.float32)
             + p_ref[1, 1, pl.ds(0, N), :].astype(jnp.float32))
    a = jnp.concatenate([left, right], axis=1)
    agg = lax.dot_general(
        a, w_ref[...], (((1,), (1,)), ((), ())),
        preferred_element_type=jnp.float32,
        precision=lax.Precision.HIGHEST,
    )
    mean = jnp.mean(agg, axis=0, keepdims=True)
    cent = agg - mean
    var = jnp.mean(cent * cent, axis=0, keepdims=True)
    inv = lax.rsqrt(var + EPSILON)
    o_ref[...] = jnp.maximum(cent * inv * g_ref[...] + b_ref[...], 0.0)


def kernel(feature, edge_index, W, gamma, beta):
    src = edge_index[0]
    dst = edge_index[1]
    npad = EPAD - E
    src_p = jnp.concatenate([src, jnp.zeros((npad,), jnp.int32)])
    dst_p = jnp.concatenate(
        [dst, N + jnp.arange(npad, dtype=jnp.int32) % (NROWS - N)])
    feat_t = jnp.zeros((2, NROWS, DH), jnp.bfloat16)
    feat_t = feat_t.at[:, :N, :].set(
        feature.reshape(N, 2, DH).transpose(1, 0, 2).astype(jnp.bfloat16))
    feat_t = feat_t.reshape(2 * NROWS, DH)
    ei = jnp.stack([src_p.reshape(16, NCHUNK, CHUNK),
                    dst_p.reshape(16, NCHUNK, CHUNK)], axis=2)

    partial = _sc_aggregate(feat_t, ei)

    out = pl.pallas_call(
        _tc_body,
        out_shape=jax.ShapeDtypeStruct((N, D), jnp.float32),
    )(partial, W, gamma.reshape(1, D), beta.reshape(1, D))
    return out
